# Initial kernel scaffold; baseline (speedup 1.0000x reference)
import functools

import jax
import jax.numpy as jnp
from jax import lax
from jax.experimental import pallas as pl
from jax.experimental.pallas import tpu as pltpu
from jax.experimental.pallas import tpu_sc as plsc

N_NODES = 10000
IN_CH = 128
HID_CH = 128
OUT_CH = 64
N_EDGES = 320000

NC = 2
NS = 16
NW = NC * NS
CH = 128
RPT = N_NODES // NS
RZ = 125

_mesh = plsc.VectorSubcoreMesh(core_axis_name="c", subcore_axis_name="s")


@functools.lru_cache(maxsize=None)
def _sc_degree():
    nchunk = N_EDGES // CH

    @functools.partial(
        pl.kernel,
        out_type=jax.ShapeDtypeStruct((NC, N_NODES, 16), jnp.float32),
        mesh=_mesh,
        scratch_types=[
            pltpu.VMEM((CH,), jnp.int32),
            pltpu.VMEM((CH, 16), jnp.float32),
            pltpu.VMEM((RPT, 16), jnp.float32),
            pltpu.VMEM_SHARED((N_NODES, 16), jnp.float32),
        ],
    )
    def degree(dst_hbm, ones_hbm, zero_hbm, out_hbm, didx, ones_v, stage, acc):
        cid = lax.axis_index("c")
        sid = lax.axis_index("s")
        wid = sid * NC + cid
        base = sid * RPT
        pltpu.sync_copy(ones_hbm, ones_v)
        pltpu.sync_copy(zero_hbm, stage)
        pltpu.sync_copy(stage, acc.at[pl.ds(base, RPT)])
        plsc.subcore_barrier()
        nmine = (nchunk - wid + NW - 1) // NW

        def body(i, carry):
            off = (wid + i * NW) * CH
            pltpu.sync_copy(dst_hbm.at[pl.ds(off, CH)], didx)
            pltpu.sync_copy(ones_v, acc.at[didx], add=True)
            return carry

        lax.fori_loop(0, nmine, body, 0)
        plsc.subcore_barrier()
        pltpu.sync_copy(acc.at[pl.ds(base, RPT)], stage)
        pltpu.sync_copy(stage, out_hbm.at[cid, pl.ds(base, RPT)])

    return degree


@functools.lru_cache(maxsize=None)
def _sc_scatter(D):
    nchunk = N_EDGES // CH

    @functools.partial(
        pl.kernel,
        out_type=jax.ShapeDtypeStruct((NC, N_NODES, D), jnp.float32),
        mesh=_mesh,
        scratch_types=[
            pltpu.VMEM((CH,), jnp.int32),
            pltpu.VMEM((CH,), jnp.int32),
            pltpu.VMEM((CH, D), jnp.float32),
            pltpu.VMEM((RZ, D), jnp.float32),
            pltpu.VMEM_SHARED((N_NODES, D), jnp.float32),
            pltpu.SemaphoreType.DMA,
        ],
    )
    def scatter(h_hbm, src_hbm, dst_hbm, zero_hbm, out_hbm,
                sidx, didx, rows, stage, acc, sem):
        cid = lax.axis_index("c")
        sid = lax.axis_index("s")
        wid = sid * NC + cid
        base = sid * RPT
        pltpu.sync_copy(zero_hbm, stage)
        for j in range(RPT // RZ):
            pltpu.sync_copy(stage, acc.at[pl.ds(base + j * RZ, RZ)])
        plsc.subcore_barrier()
        nmine = (nchunk - wid + NW - 1) // NW

        def body(i, carry):
            off = (wid + i * NW) * CH
            pltpu.sync_copy(src_hbm.at[pl.ds(off, CH)], sidx)
            pltpu.sync_copy(dst_hbm.at[pl.ds(off, CH)], didx)
            pltpu.async_copy(h_hbm.at[sidx], rows, sem).wait()
            pltpu.sync_copy(rows, acc.at[didx], add=True)
            return carry

        lax.fori_loop(0, nmine, body, 0)
        plsc.subcore_barrier()
        for j in range(RPT // RZ):
            r0 = base + j * RZ
            pltpu.sync_copy(acc.at[pl.ds(r0, RZ)], stage)
            pltpu.sync_copy(stage, out_hbm.at[cid, pl.ds(r0, RZ)])

    return scatter


BLK = 1000


def _dinv_block(d0, d1):
    deg = d0[:, :1] + d1[:, :1] + 1.0
    return lax.rsqrt(deg)


def _tc_first(x, W1, d0, d1):
    def body(x_ref, w_ref, d0_ref, d1_ref, o_ref):
        dinv = _dinv_block(d0_ref[...], d1_ref[...])
        h = jnp.dot(x_ref[...], w_ref[...], preferred_element_type=jnp.float32)
        o_ref[...] = h * dinv

    return pl.pallas_call(
        body,
        grid=(N_NODES // BLK,),
        in_specs=[
            pl.BlockSpec((BLK, IN_CH), lambda i: (i, 0)),
            pl.BlockSpec((IN_CH, HID_CH), lambda i: (0, 0)),
            pl.BlockSpec((BLK, 16), lambda i: (i, 0)),
            pl.BlockSpec((BLK, 16), lambda i: (i, 0)),
        ],
        out_specs=pl.BlockSpec((BLK, HID_CH), lambda i: (i, 0)),
        out_shape=jax.ShapeDtypeStruct((N_NODES, HID_CH), jnp.float32),
    )(x, W1, d0, d1)


def _tc_mid(p0, p1, h1p, d0, d1, b1, W2):
    def body(p0_ref, p1_ref, h_ref, d0_ref, d1_ref, b_ref, w_ref, o_ref):
        dinv = _dinv_block(d0_ref[...], d1_ref[...])
        z = (p0_ref[...] + p1_ref[...] + h_ref[...]) * dinv + b_ref[...]
        t = jnp.maximum(z, 0.0)
        o_ref[...] = jnp.dot(t, w_ref[...],
                             preferred_element_type=jnp.float32) * dinv

    return pl.pallas_call(
        body,
        grid=(N_NODES // BLK,),
        in_specs=[
            pl.BlockSpec((BLK, HID_CH), lambda i: (i, 0)),
            pl.BlockSpec((BLK, HID_CH), lambda i: (i, 0)),
            pl.BlockSpec((BLK, HID_CH), lambda i: (i, 0)),
            pl.BlockSpec((BLK, 16), lambda i: (i, 0)),
            pl.BlockSpec((BLK, 16), lambda i: (i, 0)),
            pl.BlockSpec((HID_CH,), lambda i: (0,)),
            pl.BlockSpec((HID_CH, OUT_CH), lambda i: (0, 0)),
        ],
        out_specs=pl.BlockSpec((BLK, OUT_CH), lambda i: (i, 0)),
        out_shape=jax.ShapeDtypeStruct((N_NODES, OUT_CH), jnp.float32),
    )(p0, p1, h1p, d0, d1, b1, W2)


def _tc_final(q0, q1, h2p, d0, d1, b2):
    def body(q0_ref, q1_ref, h_ref, d0_ref, d1_ref, b_ref, o_ref):
        dinv = _dinv_block(d0_ref[...], d1_ref[...])
        o_ref[...] = (q0_ref[...] + q1_ref[...] + h_ref[...]) * dinv + b_ref[...]

    return pl.pallas_call(
        body,
        grid=(N_NODES // BLK,),
        in_specs=[
            pl.BlockSpec((BLK, OUT_CH), lambda i: (i, 0)),
            pl.BlockSpec((BLK, OUT_CH), lambda i: (i, 0)),
            pl.BlockSpec((BLK, OUT_CH), lambda i: (i, 0)),
            pl.BlockSpec((BLK, 16), lambda i: (i, 0)),
            pl.BlockSpec((BLK, 16), lambda i: (i, 0)),
            pl.BlockSpec((OUT_CH,), lambda i: (0,)),
        ],
        out_specs=pl.BlockSpec((BLK, OUT_CH), lambda i: (i, 0)),
        out_shape=jax.ShapeDtypeStruct((N_NODES, OUT_CH), jnp.float32),
    )(q0, q1, h2p, d0, d1, b2)


def kernel(x, edge_index, W1, b1, W2, b2):
    ei = edge_index.astype(jnp.int32)
    src, dst = ei[0], ei[1]
    ones16 = jnp.ones((CH, 16), jnp.float32)
    zeros16 = jnp.zeros((RPT, 16), jnp.float32)
    zeros128 = jnp.zeros((RZ, HID_CH), jnp.float32)
    zeros64 = jnp.zeros((RZ, OUT_CH), jnp.float32)

    degp = _sc_degree()(dst, ones16, zeros16)
    d0, d1 = degp[0], degp[1]
    h1p = _tc_first(x, W1, d0, d1)
    P = _sc_scatter(HID_CH)(h1p, src, dst, zeros128)
    h2p = _tc_mid(P[0], P[1], h1p, d0, d1, b1, W2)
    Q = _sc_scatter(OUT_CH)(h2p, src, dst, zeros64)
    return _tc_final(Q[0], Q[1], h2p, d0, d1, b2)


# speedup vs baseline: 16.3858x; 16.3858x over previous
import functools

import jax
import jax.numpy as jnp
from jax import lax
from jax.experimental import pallas as pl
from jax.experimental.pallas import tpu as pltpu
from jax.experimental.pallas import tpu_sc as plsc

N_NODES = 10000
IN_CH = 128
HID_CH = 128
OUT_CH = 64
N_EDGES = 320000

NC = 2
NS = 16
NW = NC * NS
CH = 128
NPAD = 10240
RPT = NPAD // NS
RZ = 128

_mesh = plsc.VectorSubcoreMesh(core_axis_name="c", subcore_axis_name="s")


@functools.lru_cache(maxsize=None)
def _sc_degree():
    nchunk = N_EDGES // CH

    @functools.partial(
        pl.kernel,
        out_type=jax.ShapeDtypeStruct((NC, NPAD, 16), jnp.float32),
        mesh=_mesh,
        compiler_params=pltpu.CompilerParams(use_tc_tiling_on_sc=False),
        scratch_types=[
            pltpu.VMEM((CH,), jnp.int32),
            pltpu.VMEM((CH, 16), jnp.float32),
            pltpu.VMEM((RPT, 16), jnp.float32),
            pltpu.VMEM_SHARED((NPAD, 16), jnp.float32),
        ],
    )
    def degree(dst_hbm, ones_hbm, zero_hbm, out_hbm, didx, ones_v, stage, acc):
        cid = lax.axis_index("c")
        sid = lax.axis_index("s")
        wid = sid * NC + cid
        base = sid * RPT
        pltpu.sync_copy(ones_hbm, ones_v)
        pltpu.sync_copy(zero_hbm, stage)
        pltpu.sync_copy(stage, acc.at[pl.ds(base, RPT)])
        plsc.subcore_barrier()
        nmine = (nchunk - wid + NW - 1) // NW

        def body(i, carry):
            off = (wid + i * NW) * CH
            pltpu.sync_copy(dst_hbm.at[pl.ds(off, CH)], didx)
            pltpu.sync_copy(ones_v, acc.at[didx], add=True)
            return carry

        lax.fori_loop(0, nmine, body, 0)
        plsc.subcore_barrier()
        pltpu.sync_copy(acc.at[pl.ds(base, RPT)], stage)
        pltpu.sync_copy(stage, out_hbm.at[cid, pl.ds(base, RPT)])

    return degree


@functools.lru_cache(maxsize=None)
def _sc_scatter(D):
    nchunk = N_EDGES // CH

    @functools.partial(
        pl.kernel,
        out_type=jax.ShapeDtypeStruct((NC, NPAD, D), jnp.float32),
        mesh=_mesh,
        compiler_params=pltpu.CompilerParams(use_tc_tiling_on_sc=False),
        scratch_types=[
            pltpu.VMEM((CH,), jnp.int32),
            pltpu.VMEM((CH,), jnp.int32),
            pltpu.VMEM((CH, D), jnp.float32),
            pltpu.VMEM((RZ, D), jnp.float32),
            pltpu.VMEM_SHARED((NPAD, D), jnp.float32),
            pltpu.SemaphoreType.DMA,
        ],
    )
    def scatter(h_hbm, src_hbm, dst_hbm, zero_hbm, out_hbm,
                sidx, didx, rows, stage, acc, sem):
        cid = lax.axis_index("c")
        sid = lax.axis_index("s")
        wid = sid * NC + cid
        base = sid * RPT
        pltpu.sync_copy(zero_hbm, stage)
        for j in range(RPT // RZ):
            pltpu.sync_copy(stage, acc.at[pl.ds(base + j * RZ, RZ)])
        plsc.subcore_barrier()
        nmine = (nchunk - wid + NW - 1) // NW

        def body(i, carry):
            off = (wid + i * NW) * CH
            pltpu.sync_copy(src_hbm.at[pl.ds(off, CH)], sidx)
            pltpu.sync_copy(dst_hbm.at[pl.ds(off, CH)], didx)
            pltpu.async_copy(h_hbm.at[sidx], rows, sem).wait()
            pltpu.sync_copy(rows, acc.at[didx], add=True)
            return carry

        lax.fori_loop(0, nmine, body, 0)
        plsc.subcore_barrier()
        for j in range(RPT // RZ):
            r0 = base + j * RZ
            pltpu.sync_copy(acc.at[pl.ds(r0, RZ)], stage)
            pltpu.sync_copy(stage, out_hbm.at[cid, pl.ds(r0, RZ)])

    return scatter


BLK = 1000


def _dinv_block(d0, d1):
    deg = d0[:, :1] + d1[:, :1] + 1.0
    return lax.rsqrt(deg)


def _tc_first(x, W1, d0, d1):
    def body(x_ref, w_ref, d0_ref, d1_ref, o_ref):
        dinv = _dinv_block(d0_ref[...], d1_ref[...])
        h = jnp.dot(x_ref[...], w_ref[...], preferred_element_type=jnp.float32)
        o_ref[...] = h * dinv

    return pl.pallas_call(
        body,
        grid=(N_NODES // BLK,),
        in_specs=[
            pl.BlockSpec((BLK, IN_CH), lambda i: (i, 0)),
            pl.BlockSpec((IN_CH, HID_CH), lambda i: (0, 0)),
            pl.BlockSpec((BLK, 16), lambda i: (i, 0)),
            pl.BlockSpec((BLK, 16), lambda i: (i, 0)),
        ],
        out_specs=pl.BlockSpec((BLK, HID_CH), lambda i: (i, 0)),
        out_shape=jax.ShapeDtypeStruct((N_NODES, HID_CH), jnp.float32),
    )(x, W1, d0, d1)


def _tc_mid(p0, p1, h1p, d0, d1, b1, W2):
    def body(p0_ref, p1_ref, h_ref, d0_ref, d1_ref, b_ref, w_ref, o_ref):
        dinv = _dinv_block(d0_ref[...], d1_ref[...])
        z = (p0_ref[...] + p1_ref[...] + h_ref[...]) * dinv + b_ref[...]
        t = jnp.maximum(z, 0.0)
        o_ref[...] = jnp.dot(t, w_ref[...],
                             preferred_element_type=jnp.float32) * dinv

    return pl.pallas_call(
        body,
        grid=(N_NODES // BLK,),
        in_specs=[
            pl.BlockSpec((BLK, HID_CH), lambda i: (i, 0)),
            pl.BlockSpec((BLK, HID_CH), lambda i: (i, 0)),
            pl.BlockSpec((BLK, HID_CH), lambda i: (i, 0)),
            pl.BlockSpec((BLK, 16), lambda i: (i, 0)),
            pl.BlockSpec((BLK, 16), lambda i: (i, 0)),
            pl.BlockSpec((HID_CH,), lambda i: (0,)),
            pl.BlockSpec((HID_CH, OUT_CH), lambda i: (0, 0)),
        ],
        out_specs=pl.BlockSpec((BLK, OUT_CH), lambda i: (i, 0)),
        out_shape=jax.ShapeDtypeStruct((N_NODES, OUT_CH), jnp.float32),
    )(p0, p1, h1p, d0, d1, b1, W2)


def _tc_final(q0, q1, h2p, d0, d1, b2):
    def body(q0_ref, q1_ref, h_ref, d0_ref, d1_ref, b_ref, o_ref):
        dinv = _dinv_block(d0_ref[...], d1_ref[...])
        o_ref[...] = (q0_ref[...] + q1_ref[...] + h_ref[...]) * dinv + b_ref[...]

    return pl.pallas_call(
        body,
        grid=(N_NODES // BLK,),
        in_specs=[
            pl.BlockSpec((BLK, OUT_CH), lambda i: (i, 0)),
            pl.BlockSpec((BLK, OUT_CH), lambda i: (i, 0)),
            pl.BlockSpec((BLK, OUT_CH), lambda i: (i, 0)),
            pl.BlockSpec((BLK, 16), lambda i: (i, 0)),
            pl.BlockSpec((BLK, 16), lambda i: (i, 0)),
            pl.BlockSpec((OUT_CH,), lambda i: (0,)),
        ],
        out_specs=pl.BlockSpec((BLK, OUT_CH), lambda i: (i, 0)),
        out_shape=jax.ShapeDtypeStruct((N_NODES, OUT_CH), jnp.float32),
    )(q0, q1, h2p, d0, d1, b2)


def kernel(x, edge_index, W1, b1, W2, b2):
    ei = edge_index.astype(jnp.int32)
    src, dst = ei[0], ei[1]
    ones16 = jnp.ones((CH, 16), jnp.float32)
    zeros16 = jnp.zeros((RPT, 16), jnp.float32)
    zeros128 = jnp.zeros((RZ, HID_CH), jnp.float32)
    zeros64 = jnp.zeros((RZ, OUT_CH), jnp.float32)

    degp = _sc_degree()(dst, ones16, zeros16)
    d0, d1 = degp[0, :N_NODES], degp[1, :N_NODES]
    h1p = _tc_first(x, W1, d0, d1)
    P = _sc_scatter(HID_CH)(h1p, src, dst, zeros128)
    h2p = _tc_mid(P[0, :N_NODES], P[1, :N_NODES], h1p, d0, d1, b1, W2)
    Q = _sc_scatter(OUT_CH)(h2p, src, dst, zeros64)
    return _tc_final(Q[0, :N_NODES], Q[1, :N_NODES], h2p, d0, d1, b2)
